# baseline (speedup 1.0000x reference)
import functools

import jax
import jax.numpy as jnp
from jax import lax
from jax.experimental import pallas as pl
from jax.experimental.pallas import tpu as pltpu
from jax.experimental.pallas import tpu_sc as plsc

NC = 2
NS = 16
NW = NC * NS
LANES = 16
EPS = 0.3
N_LAYERS = 4

_mesh = lambda: plsc.VectorSubcoreMesh(core_axis_name="c", subcore_axis_name="s")
_SC_PARAMS = pltpu.CompilerParams(needs_layout_passes=False,
                                  use_tc_tiling_on_sc=False)


def _sc_deg(edge_flat, n):
    e = edge_flat.shape[0] // 2
    e_pw = e // NW
    ch = 2000

    @functools.partial(
        pl.kernel,
        out_type=jax.ShapeDtypeStruct((NW, n), jnp.float32),
        mesh=_mesh(),
        compiler_params=_SC_PARAMS,
        scratch_types=[
            pltpu.VMEM((n,), jnp.float32),
            pltpu.VMEM((ch,), jnp.int32),
        ],
    )
    def k(edge_hbm, out_hbm, deg_t, col_v):
        wid = lax.axis_index("s") * NC + lax.axis_index("c")
        base = wid * e_pw
        zero16 = jnp.zeros((LANES,), jnp.float32)
        one16 = jnp.ones((LANES,), jnp.float32)

        @pl.loop(0, n, step=LANES)
        def _(i):
            deg_t[pl.ds(i, LANES)] = zero16

        @pl.loop(0, e_pw, step=ch)
        def _(c0):
            pltpu.sync_copy(edge_hbm.at[pl.ds(e + base + c0, ch)], col_v)

            @pl.loop(0, ch, step=LANES)
            def _(i):
                idx = col_v[pl.ds(i, LANES)]
                plsc.addupdate_scatter(deg_t, [idx], one16)

        pltpu.sync_copy(deg_t, out_hbm.at[wid])

    return k(edge_flat)


NBUF = 5


def _bcast(v, lane):
    idx = jnp.full((LANES, 1), lane, jnp.int32)
    dnums = lax.GatherDimensionNumbers(
        offset_dims=(), collapsed_slice_dims=(0,), start_index_map=(0,))
    return lax.gather(v, idx, dnums, (1,),
                      mode=lax.GatherScatterMode.PROMISE_IN_BOUNDS)


def _sc_msg(h, al, ar, dis, row_r, col_r):
    n, hid = h.shape
    _, nch, ce = row_r.shape
    wb_tiles = 10
    rows_pt = n // wb_tiles
    zc = 40

    @functools.partial(
        pl.kernel,
        out_type=jax.ShapeDtypeStruct((NC, n, hid), jnp.float32),
        mesh=_mesh(),
        compiler_params=_SC_PARAMS,
        scratch_types=[
            pltpu.VMEM((n,), jnp.float32),
            pltpu.VMEM((n,), jnp.float32),
            pltpu.VMEM((n,), jnp.float32),
            pltpu.VMEM((nch, ce), jnp.int32),
            pltpu.VMEM((nch, ce), jnp.int32),
            pltpu.VMEM((ce,), jnp.float32),
            pltpu.VMEM((NBUF, ce, hid), jnp.float32),
            pltpu.VMEM_SHARED((n, hid), jnp.float32),
        ] + [pltpu.SemaphoreType.DMA] * (2 * NBUF),
    )
    def k(h_hbm, al_hbm, ar_hbm, dis_hbm, row_hbm, col_hbm, out_hbm,
          al_t, ar_t, dis_t, row_v, col_v, w_v, hr_v, acc_sh, *sems):
        sg = sems[:NBUF]
        ss = sems[NBUF:]
        cid = lax.axis_index("c")
        sid = lax.axis_index("s")
        wid = sid * NC + cid
        zero16 = jnp.zeros((LANES,), jnp.float32)

        pltpu.sync_copy(row_hbm.at[wid], row_v)
        pltpu.sync_copy(col_hbm.at[wid], col_v)
        pltpu.sync_copy(al_hbm, al_t)
        pltpu.sync_copy(ar_hbm, ar_t)
        pltpu.sync_copy(dis_hbm, dis_t)

        @pl.when(sid < wb_tiles)
        def _():
            @pl.loop(0, zc)
            def _(r):
                for d in range(hid // LANES):
                    hr_v[0, r, pl.ds(d * LANES, LANES)] = zero16

            @pl.loop(0, rows_pt, step=zc)
            def _(r0):
                pltpu.sync_copy(hr_v.at[0, pl.ds(0, zc)],
                                acc_sh.at[pl.ds(sid * rows_pt + r0, zc)])

        plsc.subcore_barrier()

        def gather_cp(j, b):
            return pltpu.make_async_copy(h_hbm.at[row_v.at[j]], hr_v.at[b],
                                         sg[b])

        def scatter_cp(j, b):
            return pltpu.make_async_copy(hr_v.at[b], acc_sh.at[col_v.at[j]],
                                         ss[b])

        @pl.loop(0, nch, step=NBUF)
        def _(j0):
            for b in range(NBUF):
                @pl.when(j0 > 0)
                def _():
                    scatter_cp(0, b).wait()

                gather_cp(j0 + b, b).start()

            for b in range(NBUF):
                j = j0 + b

                @plsc.parallel_loop(0, ce, step=LANES, unroll=2)
                def _(i):
                    r16 = row_v[j, pl.ds(i, LANES)]
                    c16 = col_v[j, pl.ds(i, LANES)]
                    t2 = (plsc.load_gather(al_t, [r16])
                          + plsc.load_gather(ar_t, [c16]))
                    t2 = t2 + t2
                    den = jnp.exp(t2) + 1.0
                    r0 = plsc.bitcast(
                        jnp.int32(0x7EF127EA) - plsc.bitcast(den, jnp.int32),
                        jnp.float32)
                    r1 = r0 * (2.0 - den * r0)
                    r2 = r1 * (2.0 - den * r1)
                    alpha = jnp.where(t2 > 80.0, 1.0, 1.0 - (r2 + r2))
                    w_v[pl.ds(i, LANES)] = (
                        alpha * plsc.load_gather(dis_t, [r16])
                        * plsc.load_gather(dis_t, [c16]))

                gather_cp(j, b).wait()

                @plsc.parallel_loop(0, ce, step=LANES, unroll=4)
                def _(i):
                    w16 = w_v[pl.ds(i, LANES)]
                    for jj in range(LANES):
                        wb = _bcast(w16, jj)
                        for d in range(hid // LANES):
                            sl = pl.ds(d * LANES, LANES)
                            hr_v[b, i + jj, sl] = hr_v[b, i + jj, sl] * wb

                pltpu.async_copy(hr_v.at[b], acc_sh.at[col_v.at[j]], ss[b],
                                 add=True)

        for b in range(NBUF):
            scatter_cp(0, b).wait()

        plsc.subcore_barrier()

        @pl.when(sid < wb_tiles)
        def _():
            pltpu.sync_copy(acc_sh.at[pl.ds(sid * rows_pt, rows_pt)],
                            out_hbm.at[cid, pl.ds(sid * rows_pt, rows_pt)])

    return k(h, al, ar, dis, row_r, col_r)


def _tc_pre(x, w1, b1, degp, wl, bl, wr, br):
    n = x.shape[0]
    hid = w1.shape[0]

    def body(x_ref, w1_ref, b1_ref, degp_ref, wl_ref, bl_ref, wr_ref, br_ref,
             h_ref, dis_ref, al_ref, ar_ref):
        h = lax.dot_general(x_ref[...], w1_ref[...], (((1,), (1,)), ((), ())))
        h = jnp.maximum(h + b1_ref[...], 0.0)
        h_ref[...] = h
        ones = jnp.ones((NW, 1), jnp.float32)
        deg = lax.dot_general(degp_ref[...], ones, (((0,), (0,)), ((), ())))
        dis_ref[...] = lax.rsqrt(deg + 1.0)
        al_ref[...] = lax.dot_general(h, wl_ref[...], (((1,), (0,)), ((), ()))) + bl_ref[...]
        ar_ref[...] = lax.dot_general(h, wr_ref[...], (((1,), (0,)), ((), ()))) + br_ref[...]

    out_shape = [
        jax.ShapeDtypeStruct((n, hid), jnp.float32),
        jax.ShapeDtypeStruct((n, 1), jnp.float32),
        jax.ShapeDtypeStruct((n, 1), jnp.float32),
        jax.ShapeDtypeStruct((n, 1), jnp.float32),
    ]
    return pl.pallas_call(body, out_shape=out_shape)(
        x, w1, b1, degp, wl, bl, wr, br)


def _tc_combine(acc, h, al, ar, dis, raw, wl, bl, wr, br):
    n, hid = h.shape

    def body(acc_ref, h_ref, al_ref, ar_ref, dis_ref, raw_ref,
             wl_ref, bl_ref, wr_ref, br_ref, hn_ref, aln_ref, arn_ref):
        dis = dis_ref[...]
        st = jnp.tanh(al_ref[...] + ar_ref[...]) * dis * dis
        hn = acc_ref[0] + acc_ref[1] + st * h_ref[...] + EPS * raw_ref[...]
        hn_ref[...] = hn
        aln_ref[...] = lax.dot_general(hn, wl_ref[...], (((1,), (0,)), ((), ()))) + bl_ref[...]
        arn_ref[...] = lax.dot_general(hn, wr_ref[...], (((1,), (0,)), ((), ()))) + br_ref[...]

    out_shape = [
        jax.ShapeDtypeStruct((n, hid), jnp.float32),
        jax.ShapeDtypeStruct((n, 1), jnp.float32),
        jax.ShapeDtypeStruct((n, 1), jnp.float32),
    ]
    return pl.pallas_call(body, out_shape=out_shape)(
        acc, h, al, ar, dis, raw, wl, bl, wr, br)


def _tc_final(acc, h, al, ar, dis, raw, w2, b2):
    n, hid = h.shape
    ncls = w2.shape[0]

    def body(acc_ref, h_ref, al_ref, ar_ref, dis_ref, raw_ref, w2_ref, b2_ref,
             out_ref):
        dis = dis_ref[...]
        st = jnp.tanh(al_ref[...] + ar_ref[...]) * dis * dis
        hn = acc_ref[0] + acc_ref[1] + st * h_ref[...] + EPS * raw_ref[...]
        logits = lax.dot_general(hn, w2_ref[...], (((1,), (1,)), ((), ()))) + b2_ref[...]
        m = jnp.max(logits, axis=1, keepdims=True)
        lse = jnp.log(jnp.sum(jnp.exp(logits - m), axis=1, keepdims=True)) + m
        out_ref[...] = logits - lse

    return pl.pallas_call(
        body, out_shape=jax.ShapeDtypeStruct((n, ncls), jnp.float32))(
        acc, h, al, ar, dis, raw, w2, b2)


def kernel(x, edge_index, W1, b1, W2, b2, att_l_w, att_l_b, att_r_w, att_r_b):
    n = x.shape[0]
    edge_flat = edge_index.reshape(-1)
    e = edge_index.shape[1]
    ce = 80
    nch = e // (NW * ce)
    row_r = edge_index[0].reshape(NW, nch, ce)
    col_r = edge_index[1].reshape(NW, nch, ce)
    degp = _sc_deg(edge_flat, n)
    h, dis, al, ar = _tc_pre(
        x, W1, b1.reshape(1, -1), degp,
        att_l_w[0].reshape(-1, 1), att_l_b[0].reshape(1, 1),
        att_r_w[0].reshape(-1, 1), att_r_b[0].reshape(1, 1))
    raw = h
    dis_flat = dis.reshape(-1)
    out = None
    for l in range(N_LAYERS):
        acc = _sc_msg(h, al.reshape(-1), ar.reshape(-1), dis_flat, row_r, col_r)
        if l + 1 < N_LAYERS:
            h, al, ar = _tc_combine(
                acc, h, al, ar, dis, raw,
                att_l_w[l + 1].reshape(-1, 1), att_l_b[l + 1].reshape(1, 1),
                att_r_w[l + 1].reshape(-1, 1), att_r_b[l + 1].reshape(1, 1))
        else:
            out = _tc_final(acc, h, al, ar, dis, raw, W2, b2.reshape(1, -1))
    return out


# speedup vs baseline: 1.2201x; 1.2201x over previous
import functools

import jax
import jax.numpy as jnp
from jax import lax
from jax.experimental import pallas as pl
from jax.experimental.pallas import tpu as pltpu
from jax.experimental.pallas import tpu_sc as plsc

NC = 2
NS = 16
NW = NC * NS
LANES = 16
EPS = 0.3
N_LAYERS = 4

_mesh = lambda: plsc.VectorSubcoreMesh(core_axis_name="c", subcore_axis_name="s")
_SC_PARAMS = pltpu.CompilerParams(needs_layout_passes=False,
                                  use_tc_tiling_on_sc=False)


def _sc_deg(edge_flat, n):
    e = edge_flat.shape[0] // 2
    e_pw = e // NW
    ch = 2000

    @functools.partial(
        pl.kernel,
        out_type=jax.ShapeDtypeStruct((NW, n), jnp.float32),
        mesh=_mesh(),
        compiler_params=_SC_PARAMS,
        scratch_types=[
            pltpu.VMEM((n,), jnp.float32),
            pltpu.VMEM((ch,), jnp.int32),
        ],
    )
    def k(edge_hbm, out_hbm, deg_t, col_v):
        wid = lax.axis_index("s") * NC + lax.axis_index("c")
        base = wid * e_pw
        zero16 = jnp.zeros((LANES,), jnp.float32)
        one16 = jnp.ones((LANES,), jnp.float32)

        @pl.loop(0, n, step=LANES)
        def _(i):
            deg_t[pl.ds(i, LANES)] = zero16

        @pl.loop(0, e_pw, step=ch)
        def _(c0):
            pltpu.sync_copy(edge_hbm.at[pl.ds(e + base + c0, ch)], col_v)

            @pl.loop(0, ch, step=LANES)
            def _(i):
                idx = col_v[pl.ds(i, LANES)]
                plsc.addupdate_scatter(deg_t, [idx], one16)

        pltpu.sync_copy(deg_t, out_hbm.at[wid])

    return k(edge_flat)


NBUF = 5


def _bcast(v, lane):
    idx = jnp.full((LANES, 1), lane, jnp.int32)
    dnums = lax.GatherDimensionNumbers(
        offset_dims=(), collapsed_slice_dims=(0,), start_index_map=(0,))
    return lax.gather(v, idx, dnums, (1,),
                      mode=lax.GatherScatterMode.PROMISE_IN_BOUNDS)


def _sc_msg(h, al, ar, dis, row_r, col_r):
    n, hid = h.shape
    _, nch, ce = row_r.shape
    wb_tiles = 10
    rows_pt = n // wb_tiles
    zc = 40

    @functools.partial(
        pl.kernel,
        out_type=jax.ShapeDtypeStruct((NC, n, hid), jnp.float32),
        mesh=_mesh(),
        compiler_params=_SC_PARAMS,
        scratch_types=[
            pltpu.VMEM((n,), jnp.float32),
            pltpu.VMEM((n,), jnp.float32),
            pltpu.VMEM((n,), jnp.float32),
            pltpu.VMEM((nch, ce), jnp.int32),
            pltpu.VMEM((nch, ce), jnp.int32),
            pltpu.VMEM((ce,), jnp.float32),
            pltpu.VMEM((NBUF, ce, hid), jnp.float32),
            pltpu.VMEM_SHARED((n, hid), jnp.float32),
        ] + [pltpu.SemaphoreType.DMA] * (2 * NBUF),
    )
    def k(h_hbm, al_hbm, ar_hbm, dis_hbm, row_hbm, col_hbm, out_hbm,
          al_t, ar_t, dis_t, row_v, col_v, w_v, hr_v, acc_sh, *sems):
        sg = sems[:NBUF]
        ss = sems[NBUF:]
        cid = lax.axis_index("c")
        sid = lax.axis_index("s")
        wid = sid * NC + cid
        zero16 = jnp.zeros((LANES,), jnp.float32)

        pltpu.sync_copy(row_hbm.at[wid], row_v)
        pltpu.sync_copy(col_hbm.at[wid], col_v)
        pltpu.sync_copy(al_hbm, al_t)
        pltpu.sync_copy(ar_hbm, ar_t)
        pltpu.sync_copy(dis_hbm, dis_t)

        @pl.when(sid < wb_tiles)
        def _():
            @pl.loop(0, zc)
            def _(r):
                for d in range(hid // LANES):
                    hr_v[0, r, pl.ds(d * LANES, LANES)] = zero16

            @pl.loop(0, rows_pt, step=zc)
            def _(r0):
                pltpu.sync_copy(hr_v.at[0, pl.ds(0, zc)],
                                acc_sh.at[pl.ds(sid * rows_pt + r0, zc)])

        plsc.subcore_barrier()

        def gather_cp(j, b):
            return pltpu.make_async_copy(h_hbm.at[row_v.at[j]], hr_v.at[b],
                                         sg[b])

        def scatter_cp(j, b):
            return pltpu.make_async_copy(hr_v.at[b], acc_sh.at[col_v.at[j]],
                                         ss[b])

        @pl.loop(0, nch, step=NBUF)
        def _(j0):
            for b in range(NBUF):
                @pl.when(j0 > 0)
                def _():
                    scatter_cp(0, b).wait()

                gather_cp(j0 + b, b).start()

            for b in range(NBUF):
                j = j0 + b

                @plsc.parallel_loop(0, ce, step=LANES, unroll=2)
                def _(i):
                    r16 = row_v[j, pl.ds(i, LANES)]
                    c16 = col_v[j, pl.ds(i, LANES)]
                    t2 = (plsc.load_gather(al_t, [r16])
                          + plsc.load_gather(ar_t, [c16]))
                    t2 = t2 + t2
                    den = jnp.exp(t2) + 1.0
                    r0 = plsc.bitcast(
                        jnp.int32(0x7EF127EA) - plsc.bitcast(den, jnp.int32),
                        jnp.float32)
                    r1 = r0 * (2.0 - den * r0)
                    r2 = r1 * (2.0 - den * r1)
                    alpha = jnp.where(t2 > 80.0, 1.0, 1.0 - (r2 + r2))
                    w_v[pl.ds(i, LANES)] = (
                        alpha * plsc.load_gather(dis_t, [r16])
                        * plsc.load_gather(dis_t, [c16]))

                gather_cp(j, b).wait()

                @plsc.parallel_loop(0, ce, step=LANES, unroll=2)
                def _(i):
                    w16 = w_v[pl.ds(i, LANES)]
                    for jj in range(LANES):
                        wb = _bcast(w16, jj)
                        for d in range(hid // LANES):
                            sl = pl.ds(d * LANES, LANES)
                            hr_v[b, i + jj, sl] = hr_v[b, i + jj, sl] * wb

                pltpu.async_copy(hr_v.at[b], acc_sh.at[col_v.at[j]], ss[b],
                                 add=True)

        for b in range(NBUF):
            scatter_cp(0, b).wait()

        plsc.subcore_barrier()

        @pl.when(sid < wb_tiles)
        def _():
            pltpu.sync_copy(acc_sh.at[pl.ds(sid * rows_pt, rows_pt)],
                            out_hbm.at[cid, pl.ds(sid * rows_pt, rows_pt)])

    return k(h, al, ar, dis, row_r, col_r)


def _tc_pre(x, w1, b1, degp, wl, bl, wr, br):
    n = x.shape[0]
    hid = w1.shape[0]

    def body(x_ref, w1_ref, b1_ref, degp_ref, wl_ref, bl_ref, wr_ref, br_ref,
             h_ref, dis_ref, al_ref, ar_ref):
        h = lax.dot_general(x_ref[...], w1_ref[...], (((1,), (1,)), ((), ())))
        h = jnp.maximum(h + b1_ref[...], 0.0)
        h_ref[...] = h
        ones = jnp.ones((NW, 1), jnp.float32)
        deg = lax.dot_general(degp_ref[...], ones, (((0,), (0,)), ((), ())))
        dis_ref[...] = lax.rsqrt(deg + 1.0)
        al_ref[...] = lax.dot_general(h, wl_ref[...], (((1,), (0,)), ((), ()))) + bl_ref[...]
        ar_ref[...] = lax.dot_general(h, wr_ref[...], (((1,), (0,)), ((), ()))) + br_ref[...]

    out_shape = [
        jax.ShapeDtypeStruct((n, hid), jnp.float32),
        jax.ShapeDtypeStruct((n, 1), jnp.float32),
        jax.ShapeDtypeStruct((n, 1), jnp.float32),
        jax.ShapeDtypeStruct((n, 1), jnp.float32),
    ]
    return pl.pallas_call(body, out_shape=out_shape)(
        x, w1, b1, degp, wl, bl, wr, br)


def _tc_combine(acc, h, al, ar, dis, raw, wl, bl, wr, br):
    n, hid = h.shape

    def body(acc_ref, h_ref, al_ref, ar_ref, dis_ref, raw_ref,
             wl_ref, bl_ref, wr_ref, br_ref, hn_ref, aln_ref, arn_ref):
        dis = dis_ref[...]
        st = jnp.tanh(al_ref[...] + ar_ref[...]) * dis * dis
        hn = acc_ref[0] + acc_ref[1] + st * h_ref[...] + EPS * raw_ref[...]
        hn_ref[...] = hn
        aln_ref[...] = lax.dot_general(hn, wl_ref[...], (((1,), (0,)), ((), ()))) + bl_ref[...]
        arn_ref[...] = lax.dot_general(hn, wr_ref[...], (((1,), (0,)), ((), ()))) + br_ref[...]

    out_shape = [
        jax.ShapeDtypeStruct((n, hid), jnp.float32),
        jax.ShapeDtypeStruct((n, 1), jnp.float32),
        jax.ShapeDtypeStruct((n, 1), jnp.float32),
    ]
    return pl.pallas_call(body, out_shape=out_shape)(
        acc, h, al, ar, dis, raw, wl, bl, wr, br)


def _tc_final(acc, h, al, ar, dis, raw, w2, b2):
    n, hid = h.shape
    ncls = w2.shape[0]

    def body(acc_ref, h_ref, al_ref, ar_ref, dis_ref, raw_ref, w2_ref, b2_ref,
             out_ref):
        dis = dis_ref[...]
        st = jnp.tanh(al_ref[...] + ar_ref[...]) * dis * dis
        hn = acc_ref[0] + acc_ref[1] + st * h_ref[...] + EPS * raw_ref[...]
        logits = lax.dot_general(hn, w2_ref[...], (((1,), (1,)), ((), ()))) + b2_ref[...]
        m = jnp.max(logits, axis=1, keepdims=True)
        lse = jnp.log(jnp.sum(jnp.exp(logits - m), axis=1, keepdims=True)) + m
        out_ref[...] = logits - lse

    return pl.pallas_call(
        body, out_shape=jax.ShapeDtypeStruct((n, ncls), jnp.float32))(
        acc, h, al, ar, dis, raw, w2, b2)


def kernel(x, edge_index, W1, b1, W2, b2, att_l_w, att_l_b, att_r_w, att_r_b):
    n = x.shape[0]
    edge_flat = edge_index.reshape(-1)
    e = edge_index.shape[1]
    ce = 80
    nch = e // (NW * ce)
    row_r = edge_index[0].reshape(NW, nch, ce)
    col_r = edge_index[1].reshape(NW, nch, ce)
    degp = _sc_deg(edge_flat, n)
    h, dis, al, ar = _tc_pre(
        x, W1, b1.reshape(1, -1), degp,
        att_l_w[0].reshape(-1, 1), att_l_b[0].reshape(1, 1),
        att_r_w[0].reshape(-1, 1), att_r_b[0].reshape(1, 1))
    raw = h
    dis_flat = dis.reshape(-1)
    out = None
    for l in range(N_LAYERS):
        acc = _sc_msg(h, al.reshape(-1), ar.reshape(-1), dis_flat, row_r, col_r)
        if l + 1 < N_LAYERS:
            h, al, ar = _tc_combine(
                acc, h, al, ar, dis, raw,
                att_l_w[l + 1].reshape(-1, 1), att_l_b[l + 1].reshape(1, 1),
                att_r_w[l + 1].reshape(-1, 1), att_r_b[l + 1].reshape(1, 1))
        else:
            out = _tc_final(acc, h, al, ar, dis, raw, W2, b2.reshape(1, -1))
    return out
